# baseline (speedup 1.0000x reference)
import functools

import jax
import jax.numpy as jnp
from jax import lax
from jax.experimental import pallas as pl
from jax.experimental.pallas import tpu as pltpu
from jax.experimental.pallas import tpu_sc as plsc

_N_NODES = 10000
_N_EDGES = 320000
_H = 128
_NC = 2
_NS = 16
_NW = _NC * _NS
_L = 16
_EPW = _N_EDGES // _NW

_B = 2560
_NB = _N_EDGES // _B


def _sc_gather_types(x, i, j):
    mesh = plsc.VectorSubcoreMesh(
        core_axis_name="c", subcore_axis_name="s",
        num_cores=_NC, num_subcores=_NS)

    @functools.partial(
        pl.kernel,
        out_type=(jax.ShapeDtypeStruct((_N_EDGES,), jnp.int32),
                  jax.ShapeDtypeStruct((_N_EDGES,), jnp.int32)),
        mesh=mesh,
        scratch_types=[
            pltpu.VMEM((_N_NODES,), jnp.int32),
            pltpu.VMEM((_EPW,), jnp.int32),
            pltpu.VMEM((_EPW,), jnp.int32),
        ],
        compiler_params=pltpu.CompilerParams(needs_layout_passes=False),
    )
    def sc_kernel(x_hbm, i_hbm, j_hbm, xi_hbm, xj_hbm, x_v, idx_v, out_v):
        wid = lax.axis_index("s") * _NC + lax.axis_index("c")
        base = wid * _EPW
        pltpu.sync_copy(x_hbm, x_v)
        for src, dst in ((i_hbm, xi_hbm), (j_hbm, xj_hbm)):
            pltpu.sync_copy(src.at[pl.ds(base, _EPW)], idx_v)

            def body(k, _):
                sl = pl.ds(k * _L, _L)
                out_v[sl] = plsc.load_gather(x_v, [idx_v[sl]])
                return 0

            lax.fori_loop(0, _EPW // _L, body, 0, unroll=8)
            pltpu.sync_copy(out_v, dst.at[pl.ds(base, _EPW)])

    return sc_kernel(x, i, j)


def _tc_body(rbf_ref, xi_ref, xj_ref, emb_ref, wr_ref, br_ref, wl_ref,
             bl_ref, out_ref, t_ref):
    @pl.when(pl.program_id(0) == 0)
    def _():
        embp = jnp.concatenate(
            [emb_ref[...], jnp.zeros((_H - 100, _H), jnp.float32)], axis=0)
        t_ref[0:_H, :] = jnp.dot(embp, wl_ref[0:_H, :],
                                 preferred_element_type=jnp.float32)
        t_ref[_H:2 * _H, :] = jnp.dot(embp, wl_ref[_H:2 * _H, :],
                                      preferred_element_type=jnp.float32)

    r = jax.nn.silu(jnp.dot(rbf_ref[...], wr_ref[...],
                            preferred_element_type=jnp.float32) + br_ref[...])
    cio = lax.broadcasted_iota(jnp.int32, (_B, _H), 1)
    ohi = jnp.where(xi_ref[...] == cio, 1.0, 0.0)
    ohj = jnp.where(xj_ref[...] == cio, 1.0, 0.0)
    acc = jnp.dot(ohi, t_ref[0:_H, :], preferred_element_type=jnp.float32)
    acc += jnp.dot(ohj, t_ref[_H:2 * _H, :], preferred_element_type=jnp.float32)
    acc += jnp.dot(r, wl_ref[2 * _H:3 * _H, :],
                   preferred_element_type=jnp.float32)
    out_ref[...] = jax.nn.silu(acc + bl_ref[...])


def _tc_fused(rbf8, xi3, xj3, emb, wr8, br, wl, bl):
    full = lambda shape: pl.BlockSpec(shape, lambda b: (0,) * len(shape))
    return pl.pallas_call(
        _tc_body,
        grid=(_NB,),
        in_specs=[
            pl.BlockSpec((_B, 8), lambda b: (b, 0)),
            pl.BlockSpec((_B, 1), lambda b: (b, 0)),
            pl.BlockSpec((_B, 1), lambda b: (b, 0)),
            full((100, _H)),
            full((8, _H)),
            full((1, _H)),
            full((3 * _H, _H)),
            full((1, _H)),
        ],
        out_specs=pl.BlockSpec((_B, _H), lambda b: (b, 0)),
        out_shape=jax.ShapeDtypeStruct((_N_EDGES, _H), jnp.float32),
        scratch_shapes=[pltpu.VMEM((2 * _H, _H), jnp.float32)],
        compiler_params=pltpu.CompilerParams(
            dimension_semantics=("arbitrary",)),
    )(rbf8, xi3, xj3, emb, wr8, br, wl, bl)


def kernel(x, rbf, i, j, emb, W_rbf, b_rbf, W_lin, b_lin):
    xi, xj = _sc_gather_types(x, i, j)
    rbf8 = jnp.concatenate(
        [rbf, jnp.zeros((_N_EDGES, 2), rbf.dtype)], axis=1)
    wr8 = jnp.concatenate([W_rbf, jnp.zeros((2, _H), W_rbf.dtype)], axis=0)
    return _tc_fused(rbf8,
                     xi.reshape(_N_EDGES, 1), xj.reshape(_N_EDGES, 1),
                     emb, wr8, b_rbf.reshape(1, _H), W_lin,
                     b_lin.reshape(1, _H))


# speedup vs baseline: 2.2034x; 2.2034x over previous
import functools

import jax
import jax.numpy as jnp
from jax import lax
from jax.experimental import pallas as pl
from jax.experimental.pallas import tpu as pltpu
from jax.experimental.pallas import tpu_sc as plsc

_N_NODES = 10000
_N_EDGES = 320000
_H = 128
_NC = 2
_NS = 16
_NW = _NC * _NS
_L = 16
_EPW = _N_EDGES // _NW

_B = 2560
_NB = _N_EDGES // _B


def _sc_gather_types(x, i, j):
    mesh = plsc.VectorSubcoreMesh(
        core_axis_name="c", subcore_axis_name="s",
        num_cores=_NC, num_subcores=_NS)

    @functools.partial(
        pl.kernel,
        out_type=(jax.ShapeDtypeStruct((_N_EDGES,), jnp.int32),
                  jax.ShapeDtypeStruct((_N_EDGES,), jnp.int32)),
        mesh=mesh,
        scratch_types=[
            pltpu.VMEM((_N_NODES,), jnp.int32),
            pltpu.VMEM((_EPW,), jnp.int32),
            pltpu.VMEM((_EPW,), jnp.int32),
        ],
        compiler_params=pltpu.CompilerParams(needs_layout_passes=False),
    )
    def sc_kernel(x_hbm, i_hbm, j_hbm, xi_hbm, xj_hbm, x_v, idx_v, out_v):
        wid = lax.axis_index("s") * _NC + lax.axis_index("c")
        base = wid * _EPW
        pltpu.sync_copy(x_hbm, x_v)
        for src, dst in ((i_hbm, xi_hbm), (j_hbm, xj_hbm)):
            pltpu.sync_copy(src.at[pl.ds(base, _EPW)], idx_v)

            @plsc.parallel_loop(0, _EPW // _L, unroll=8)
            def body(k):
                sl = pl.ds(k * _L, _L)
                out_v[sl] = plsc.load_gather(x_v, [idx_v[sl]])

            pltpu.sync_copy(out_v, dst.at[pl.ds(base, _EPW)])

    return sc_kernel(x, i, j)


def _silu(v):
    t = v * 0.5
    return t * jnp.tanh(t) + t


def _tc_body(rbf_ref, xi_ref, xj_ref, emb_ref, wr_ref, br_ref, wl_ref,
             bl_ref, out_ref, t_ref):
    @pl.when(pl.program_id(0) == 0)
    def _():
        embp = jnp.concatenate(
            [emb_ref[...], jnp.zeros((_H - 100, _H), jnp.float32)], axis=0)
        t_ref[0:_H, :] = jnp.dot(embp, wl_ref[0:_H, :],
                                 preferred_element_type=jnp.float32)
        t_ref[_H:2 * _H, :] = jnp.dot(embp, wl_ref[_H:2 * _H, :],
                                      preferred_element_type=jnp.float32)

    r = _silu(jnp.dot(rbf_ref[...], wr_ref[...],
                      preferred_element_type=jnp.float32) + br_ref[...])
    cio = lax.broadcasted_iota(jnp.int32, (_H, _B), 0)
    ohi = jnp.where(xi_ref[...].reshape(1, _B) == cio, 1.0, 0.0)
    ohj = jnp.where(xj_ref[...].reshape(1, _B) == cio, 1.0, 0.0)
    dn = (((0,), (0,)), ((), ()))
    acc = lax.dot_general(ohi, t_ref[0:_H, :], dn,
                          preferred_element_type=jnp.float32)
    acc += lax.dot_general(ohj, t_ref[_H:2 * _H, :], dn,
                           preferred_element_type=jnp.float32)
    acc += jnp.dot(r, wl_ref[2 * _H:3 * _H, :],
                   preferred_element_type=jnp.float32)
    out_ref[...] = _silu(acc + bl_ref[...])


def _tc_fused(rbf, xi, xj, emb, wr, br, wl, bl):
    full = lambda shape: pl.BlockSpec(shape, lambda b: (0,) * len(shape))
    return pl.pallas_call(
        _tc_body,
        grid=(_NB,),
        in_specs=[
            pl.BlockSpec((_B, 6), lambda b: (b, 0)),
            pl.BlockSpec((1, _B // _H, _H), lambda b: (b, 0, 0)),
            pl.BlockSpec((1, _B // _H, _H), lambda b: (b, 0, 0)),
            full((100, _H)),
            full((6, _H)),
            full((1, _H)),
            full((3 * _H, _H)),
            full((1, _H)),
        ],
        out_specs=pl.BlockSpec((_B, _H), lambda b: (b, 0)),
        out_shape=jax.ShapeDtypeStruct((_N_EDGES, _H), jnp.float32),
        scratch_shapes=[pltpu.VMEM((2 * _H, _H), jnp.float32)],
        compiler_params=pltpu.CompilerParams(
            dimension_semantics=("arbitrary",)),
    )(rbf, xi, xj, emb, wr, br, wl, bl)


def kernel(x, rbf, i, j, emb, W_rbf, b_rbf, W_lin, b_lin):
    xi, xj = _sc_gather_types(x, i, j)
    xi2 = xi.reshape(_NB, _B // _H, _H)
    xj2 = xj.reshape(_NB, _B // _H, _H)
    return _tc_fused(rbf, xi2, xj2, emb, W_rbf, b_rbf.reshape(1, _H), W_lin,
                     b_lin.reshape(1, _H))
